# baseline (speedup 1.0000x reference)
import jax
import jax.numpy as jnp
import numpy as np
from jax import lax
from jax.experimental import pallas as pl
from jax.experimental.pallas import tpu as pltpu
from jax.experimental.pallas import tpu_sc as plsc

ENTITIES = 100000
RELATIONS = 100000
DIM = 64
B = 4096
L = 200
MAX_INPUTS_LENGTH = 200

NC = 2
NS = 16
NW = NC * NS
LANES = 16

FLAT = B * L
PER_W = FLAT // NW
CHUNK = 400
NCHUNK = PER_W // CHUNK

ROWD = L * DIM
BB = 512
RB = 512


def _fourier_pe(max_len, dim):
    input_positions = np.arange(max_len).reshape((-1, 1))
    embedding_positions = np.arange(dim).reshape((1, -1))
    relative = 2.0 * (embedding_positions // 2) / dim
    angles = input_positions / np.power(10000, relative)
    pe = np.zeros(angles.shape)
    pe[:, 0::2] = np.sin(angles[:, 0::2])
    pe[:, 1::2] = np.cos(angles[:, 1::2])
    return pe.astype(np.float32)


_PE_COL = _fourier_pe(MAX_INPUTS_LENGTH, DIM).reshape(ROWD, 1)


def _sc_body(merged_hbm, ids_hbm, types_hbm, out_hbm,
             idx_v, types_v, rows0, rows1, sg0, sg1, so0, so1):
    wid = lax.axis_index("s") * NC + lax.axis_index("c")
    base_w = wid * PER_W
    rows = (rows0, rows1)
    sg = (sg0, sg1)
    so = (so0, so1)

    pltpu.sync_copy(ids_hbm.at[pl.ds(base_w, PER_W)], idx_v)
    pltpu.sync_copy(types_hbm.at[pl.ds(base_w, PER_W)], types_v)

    def idx_body(i, c):
        s = pl.ds(i * LANES, LANES)
        idx_v[s] = idx_v[s] + types_v[s] * ENTITIES
        return c
    lax.fori_loop(0, PER_W // LANES, idx_body, 0, unroll=4)

    def gather_start(g, b):
        pltpu.async_copy(
            merged_hbm.at[idx_v.at[pl.ds(g * CHUNK, CHUNK)]], rows[b], sg[b])

    def gather_wait(g, b):
        pltpu.make_async_copy(
            merged_hbm.at[idx_v.at[pl.ds(g * CHUNK, CHUNK)]], rows[b], sg[b]
        ).wait()

    def write_start(g, b):
        pltpu.async_copy(
            rows[b], out_hbm.at[pl.ds(base_w + g * CHUNK, CHUNK)], so[b])

    def write_wait(g, b):
        pltpu.make_async_copy(
            rows[b], out_hbm.at[pl.ds(base_w + g * CHUNK, CHUNK)], so[b]
        ).wait()

    gather_start(0, 0)

    def outer(g2, c):
        for bi in range(2):
            g = g2 * 2 + bi
            gather_wait(g, bi)
            @pl.when(g >= 1)
            def _():
                write_wait(g - 1, 1 - bi)

            @pl.when(g + 1 < NCHUNK)
            def _():
                gather_start(g + 1, 1 - bi)

            write_start(g, bi)
        return c

    lax.fori_loop(0, NCHUNK // 2, outer, 0)
    write_wait(NCHUNK - 1, 1)


def _tc_body(in_ref, pe_ref, out_ref):
    out_ref[...] = jnp.transpose(in_ref[...], (1, 0)) + pe_ref[...]


def kernel(object_ids, object_types, entity_embeddings, relation_embeddings):
    merged = jnp.concatenate([entity_embeddings, relation_embeddings], axis=0)
    ids = object_ids.reshape(-1).astype(jnp.int32)
    types = object_types.reshape(-1).astype(jnp.int32)
    pe_col = jnp.asarray(_PE_COL)

    mesh = plsc.VectorSubcoreMesh(core_axis_name="c", subcore_axis_name="s")
    run_sc = pl.kernel(
        _sc_body,
        out_type=jax.ShapeDtypeStruct((FLAT, DIM), jnp.float32),
        mesh=mesh,
        scratch_types=[
            pltpu.VMEM((PER_W,), jnp.int32),
            pltpu.VMEM((PER_W,), jnp.int32),
            pltpu.VMEM((CHUNK, DIM), jnp.float32),
            pltpu.VMEM((CHUNK, DIM), jnp.float32),
            pltpu.SemaphoreType.DMA,
            pltpu.SemaphoreType.DMA,
            pltpu.SemaphoreType.DMA,
            pltpu.SemaphoreType.DMA,
        ],
        compiler_params=pltpu.CompilerParams(use_tc_tiling_on_sc=False),
    )
    gathered = run_sc(merged, ids, types)

    in2 = gathered.reshape(B, ROWD)
    out2 = pl.pallas_call(
        _tc_body,
        grid=(B // BB, ROWD // RB),
        in_specs=[
            pl.BlockSpec((BB, RB), lambda i, j: (i, j)),
            pl.BlockSpec((RB, 1), lambda i, j: (j, 0)),
        ],
        out_specs=pl.BlockSpec((RB, BB), lambda i, j: (j, i)),
        out_shape=jax.ShapeDtypeStruct((ROWD, B), jnp.float32),
    )(in2, pe_col)

    return out2.reshape(L, DIM, B).transpose(2, 0, 1)


# speedup vs baseline: 1.3766x; 1.3766x over previous
import jax
import jax.numpy as jnp
import numpy as np
from jax import lax
from jax.experimental import pallas as pl
from jax.experimental.pallas import tpu as pltpu
from jax.experimental.pallas import tpu_sc as plsc

ENTITIES = 100000
RELATIONS = 100000
DIM = 64
B = 4096
L = 200
MAX_INPUTS_LENGTH = 200

NC = 2
NS = 16
NW = NC * NS
LANES = 16

FLAT = B * L
PER_W = FLAT // NW
CHUNK = 400
NCHUNK = PER_W // CHUNK

ROWD = L * DIM
BB = 512
RB = 512


def _fourier_pe(max_len, dim):
    input_positions = np.arange(max_len).reshape((-1, 1))
    embedding_positions = np.arange(dim).reshape((1, -1))
    relative = 2.0 * (embedding_positions // 2) / dim
    angles = input_positions / np.power(10000, relative)
    pe = np.zeros(angles.shape)
    pe[:, 0::2] = np.sin(angles[:, 0::2])
    pe[:, 1::2] = np.cos(angles[:, 1::2])
    return pe.astype(np.float32)


_PE_COL = _fourier_pe(MAX_INPUTS_LENGTH, DIM).reshape(ROWD, 1)


def _sc_body(merged_hbm, ids_hbm, types_hbm, out_hbm,
             idx_v, types_v, rows0, rows1, sg0, sg1, so0, so1):
    wid = lax.axis_index("s") * NC + lax.axis_index("c")
    base_w = wid * PER_W
    rows = (rows0, rows1)
    sg = (sg0, sg1)
    so = (so0, so1)

    pltpu.sync_copy(ids_hbm.at[pl.ds(base_w, PER_W)], idx_v)
    pltpu.sync_copy(types_hbm.at[pl.ds(base_w, PER_W)], types_v)

    def idx_body(i, c):
        s = pl.ds(i * LANES, LANES)
        idx_v[s] = idx_v[s] + types_v[s] * ENTITIES
        return c
    lax.fori_loop(0, PER_W // LANES, idx_body, 0, unroll=4)

    def gather_start(g, b):
        pltpu.async_copy(
            merged_hbm.at[idx_v.at[pl.ds(g * CHUNK, CHUNK)]], rows[b], sg[b])

    def gather_wait(g, b):
        pltpu.make_async_copy(
            merged_hbm.at[idx_v.at[pl.ds(g * CHUNK, CHUNK)]], rows[b], sg[b]
        ).wait()

    def write_start(g, b):
        pltpu.async_copy(
            rows[b], out_hbm.at[pl.ds(base_w + g * CHUNK, CHUNK)], so[b])

    def write_wait(g, b):
        pltpu.make_async_copy(
            rows[b], out_hbm.at[pl.ds(base_w + g * CHUNK, CHUNK)], so[b]
        ).wait()

    gather_start(0, 0)

    def outer(g2, c):
        for bi in range(2):
            g = g2 * 2 + bi
            gather_wait(g, bi)
            @pl.when(g >= 1)
            def _():
                write_wait(g - 1, 1 - bi)

            @pl.when(g + 1 < NCHUNK)
            def _():
                gather_start(g + 1, 1 - bi)

            write_start(g, bi)
        return c

    lax.fori_loop(0, NCHUNK // 2, outer, 0)
    write_wait(NCHUNK - 1, 1)


TBB = 128


def _tc_body(in_ref, pe_ref, out_ref):
    x = in_ref[...]
    y = x.reshape(TBB, L // 2, 128)
    z = jnp.transpose(y, (1, 2, 0))
    out_ref[...] = z.reshape(ROWD, TBB) + pe_ref[...]


def kernel(object_ids, object_types, entity_embeddings, relation_embeddings):
    merged = jnp.concatenate([entity_embeddings, relation_embeddings], axis=0)
    ids = object_ids.reshape(-1).astype(jnp.int32)
    types = object_types.reshape(-1).astype(jnp.int32)
    pe_col = jnp.asarray(_PE_COL)

    mesh = plsc.VectorSubcoreMesh(core_axis_name="c", subcore_axis_name="s")
    run_sc = pl.kernel(
        _sc_body,
        out_type=jax.ShapeDtypeStruct((FLAT, DIM), jnp.float32),
        mesh=mesh,
        scratch_types=[
            pltpu.VMEM((PER_W,), jnp.int32),
            pltpu.VMEM((PER_W,), jnp.int32),
            pltpu.VMEM((CHUNK, DIM), jnp.float32),
            pltpu.VMEM((CHUNK, DIM), jnp.float32),
            pltpu.SemaphoreType.DMA,
            pltpu.SemaphoreType.DMA,
            pltpu.SemaphoreType.DMA,
            pltpu.SemaphoreType.DMA,
        ],
        compiler_params=pltpu.CompilerParams(use_tc_tiling_on_sc=False),
    )
    gathered = run_sc(merged, ids, types)

    in2 = gathered.reshape(FLAT // 2, 128)
    out2 = pl.pallas_call(
        _tc_body,
        grid=(B // TBB,),
        in_specs=[
            pl.BlockSpec((TBB * L // 2, 128), lambda i: (i, 0)),
            pl.BlockSpec((ROWD, 1), lambda i: (0, 0)),
        ],
        out_specs=pl.BlockSpec((ROWD, TBB), lambda i: (0, i)),
        out_shape=jax.ShapeDtypeStruct((ROWD, B), jnp.float32),
    )(in2, pe_col)

    return out2.reshape(L, DIM, B).transpose(2, 0, 1)


# speedup vs baseline: 1.5741x; 1.1435x over previous
import jax
import jax.numpy as jnp
import numpy as np
from jax import lax
from jax.experimental import pallas as pl
from jax.experimental.pallas import tpu as pltpu
from jax.experimental.pallas import tpu_sc as plsc

ENTITIES = 100000
RELATIONS = 100000
DIM = 64
B = 4096
L = 200
MAX_INPUTS_LENGTH = 200

NC = 2
NS = 16
NW = NC * NS
LANES = 16

FLAT = B * L
PER_W = FLAT // NW
CHUNK = 400
NCHUNK = PER_W // CHUNK

ROWD = L * DIM
BB = 512
RB = 512


def _fourier_pe(max_len, dim):
    input_positions = np.arange(max_len).reshape((-1, 1))
    embedding_positions = np.arange(dim).reshape((1, -1))
    relative = 2.0 * (embedding_positions // 2) / dim
    angles = input_positions / np.power(10000, relative)
    pe = np.zeros(angles.shape)
    pe[:, 0::2] = np.sin(angles[:, 0::2])
    pe[:, 1::2] = np.cos(angles[:, 1::2])
    return pe.astype(np.float32)


_PE_COL = _fourier_pe(MAX_INPUTS_LENGTH, DIM).reshape(ROWD, 1)


def _sc_body(merged_hbm, ids_hbm, types_hbm, out_hbm,
             idx_v, types_v, rows0, rows1, sg0, sg1, so0, so1):
    wid = lax.axis_index("s") * NC + lax.axis_index("c")
    base_w = wid * PER_W
    rows = (rows0, rows1)
    sg = (sg0, sg1)
    so = (so0, so1)

    pltpu.sync_copy(ids_hbm.at[pl.ds(base_w, PER_W)], idx_v)
    pltpu.sync_copy(types_hbm.at[pl.ds(base_w, PER_W)], types_v)

    def idx_body(i, c):
        s = pl.ds(i * LANES, LANES)
        idx_v[s] = idx_v[s] + types_v[s] * ENTITIES
        return c
    lax.fori_loop(0, PER_W // LANES, idx_body, 0, unroll=4)

    def gather_start(g, b):
        pltpu.async_copy(
            merged_hbm.at[idx_v.at[pl.ds(g * CHUNK, CHUNK)]], rows[b], sg[b])

    def gather_wait(g, b):
        pltpu.make_async_copy(
            merged_hbm.at[idx_v.at[pl.ds(g * CHUNK, CHUNK)]], rows[b], sg[b]
        ).wait()

    def write_start(g, b):
        pltpu.async_copy(
            rows[b], out_hbm.at[pl.ds(base_w + g * CHUNK, CHUNK)], so[b])

    def write_wait(g, b):
        pltpu.make_async_copy(
            rows[b], out_hbm.at[pl.ds(base_w + g * CHUNK, CHUNK)], so[b]
        ).wait()

    gather_start(0, 0)

    def outer(g2, c):
        for bi in range(2):
            g = g2 * 2 + bi
            gather_wait(g, bi)
            @pl.when(g >= 1)
            def _():
                write_wait(g - 1, 1 - bi)

            @pl.when(g + 1 < NCHUNK)
            def _():
                gather_start(g + 1, 1 - bi)

            write_start(g, bi)
        return c

    lax.fori_loop(0, NCHUNK // 2, outer, 0)
    write_wait(NCHUNK - 1, 1)


TBB = 128


def _tc_body(in_ref, pe_ref, out_ref):
    x = in_ref[...]
    y = x.reshape(TBB, L // 2, 128)
    eye = jax.lax.broadcasted_iota(jnp.int32, (TBB, TBB), 0) == \
        jax.lax.broadcasted_iota(jnp.int32, (TBB, TBB), 1)
    z = jax.lax.dot_general(
        y, eye.astype(jnp.float32),
        dimension_numbers=(((0,), (0,)), ((), ())),
        preferred_element_type=jnp.float32)
    out_ref[...] = z.reshape(ROWD, TBB) + pe_ref[...]


def kernel(object_ids, object_types, entity_embeddings, relation_embeddings):
    merged = jnp.concatenate([entity_embeddings, relation_embeddings], axis=0)
    ids = object_ids.reshape(-1).astype(jnp.int32)
    types = object_types.reshape(-1).astype(jnp.int32)
    pe_col = jnp.asarray(_PE_COL)

    mesh = plsc.VectorSubcoreMesh(core_axis_name="c", subcore_axis_name="s")
    run_sc = pl.kernel(
        _sc_body,
        out_type=jax.ShapeDtypeStruct((FLAT, DIM), jnp.float32),
        mesh=mesh,
        scratch_types=[
            pltpu.VMEM((PER_W,), jnp.int32),
            pltpu.VMEM((PER_W,), jnp.int32),
            pltpu.VMEM((CHUNK, DIM), jnp.float32),
            pltpu.VMEM((CHUNK, DIM), jnp.float32),
            pltpu.SemaphoreType.DMA,
            pltpu.SemaphoreType.DMA,
            pltpu.SemaphoreType.DMA,
            pltpu.SemaphoreType.DMA,
        ],
        compiler_params=pltpu.CompilerParams(use_tc_tiling_on_sc=False),
    )
    gathered = run_sc(merged, ids, types)

    in2 = gathered.reshape(FLAT // 2, 128)
    out2 = pl.pallas_call(
        _tc_body,
        grid=(B // TBB,),
        in_specs=[
            pl.BlockSpec((TBB * L // 2, 128), lambda i: (i, 0)),
            pl.BlockSpec((ROWD, 1), lambda i: (0, 0)),
        ],
        out_specs=pl.BlockSpec((ROWD, TBB), lambda i: (0, i)),
        out_shape=jax.ShapeDtypeStruct((ROWD, B), jnp.float32),
    )(in2, pe_col)

    return out2.reshape(L, DIM, B).transpose(2, 0, 1)


# speedup vs baseline: 1.5996x; 1.0162x over previous
import jax
import jax.numpy as jnp
import numpy as np
from jax import lax
from jax.experimental import pallas as pl
from jax.experimental.pallas import tpu as pltpu
from jax.experimental.pallas import tpu_sc as plsc

ENTITIES = 100000
RELATIONS = 100000
DIM = 64
B = 4096
L = 200
MAX_INPUTS_LENGTH = 200

NC = 2
NS = 16
NW = NC * NS
LANES = 16

FLAT = B * L
PER_W = FLAT // NW
CHUNK = 400
NCHUNK = PER_W // CHUNK

ROWD = L * DIM
BB = 512
RB = 512


def _fourier_pe(max_len, dim):
    input_positions = np.arange(max_len).reshape((-1, 1))
    embedding_positions = np.arange(dim).reshape((1, -1))
    relative = 2.0 * (embedding_positions // 2) / dim
    angles = input_positions / np.power(10000, relative)
    pe = np.zeros(angles.shape)
    pe[:, 0::2] = np.sin(angles[:, 0::2])
    pe[:, 1::2] = np.cos(angles[:, 1::2])
    return pe.astype(np.float32)


_PE_COL = _fourier_pe(MAX_INPUTS_LENGTH, DIM).reshape(ROWD, 1)


def _sc_body(merged_hbm, ids_hbm, types_hbm, out_hbm,
             idx_v, types_v, rows0, rows1, sg0, sg1, so0, so1):
    wid = lax.axis_index("s") * NC + lax.axis_index("c")
    base_w = wid * PER_W
    rows = (rows0, rows1)
    sg = (sg0, sg1)
    so = (so0, so1)

    pltpu.sync_copy(ids_hbm.at[pl.ds(base_w, PER_W)], idx_v)
    pltpu.sync_copy(types_hbm.at[pl.ds(base_w, PER_W)], types_v)

    def idx_body(i, c):
        s = pl.ds(i * LANES, LANES)
        idx_v[s] = idx_v[s] + types_v[s] * ENTITIES
        return c
    lax.fori_loop(0, PER_W // LANES, idx_body, 0, unroll=4)

    def gather_start(g, b):
        pltpu.async_copy(
            merged_hbm.at[idx_v.at[pl.ds(g * CHUNK, CHUNK)]], rows[b], sg[b])

    def gather_wait(g, b):
        pltpu.make_async_copy(
            merged_hbm.at[idx_v.at[pl.ds(g * CHUNK, CHUNK)]], rows[b], sg[b]
        ).wait()

    def write_start(g, b):
        pltpu.async_copy(
            rows[b], out_hbm.at[pl.ds(base_w + g * CHUNK, CHUNK)], so[b])

    def write_wait(g, b):
        pltpu.make_async_copy(
            rows[b], out_hbm.at[pl.ds(base_w + g * CHUNK, CHUNK)], so[b]
        ).wait()

    gather_start(0, 0)

    def outer(g2, c):
        for bi in range(2):
            g = g2 * 2 + bi
            gather_wait(g, bi)
            @pl.when(g >= 1)
            def _():
                write_wait(g - 1, 1 - bi)

            @pl.when(g + 1 < NCHUNK)
            def _():
                gather_start(g + 1, 1 - bi)

            write_start(g, bi)
        return c

    lax.fori_loop(0, NCHUNK // 2, outer, 0)
    write_wait(NCHUNK - 1, 1)


TBB = 128
MRB = 2000
IRB = 128


def _pack_tables_body(e_ref, r_ref, m_ref):
    i = pl.program_id(0)

    def pack(x):
        y = x.reshape(MRB, 2, DIM)
        return jnp.concatenate([y[:, 0, :], y[:, 1, :]], axis=-1)

    @pl.when(i < (ENTITIES // 2) // MRB)
    def _():
        m_ref[...] = pack(e_ref[...])

    @pl.when(i >= (ENTITIES // 2) // MRB)
    def _():
        m_ref[...] = pack(r_ref[...])


def _tc_body(in_ref, pe_ref, out_ref):
    x = in_ref[...]
    y = x.reshape(TBB, L // 2, 128)
    eye = jax.lax.broadcasted_iota(jnp.int32, (TBB, TBB), 0) == \
        jax.lax.broadcasted_iota(jnp.int32, (TBB, TBB), 1)
    z = jax.lax.dot_general(
        y, eye.astype(jnp.float32),
        dimension_numbers=(((0,), (0,)), ((), ())),
        preferred_element_type=jnp.float32)
    out_ref[...] = z.reshape(ROWD, TBB) + pe_ref[...]


def kernel(object_ids, object_types, entity_embeddings, relation_embeddings):
    pe_col = jnp.asarray(_PE_COL)

    nhalf = (ENTITIES // 2) // MRB
    m2 = pl.pallas_call(
        _pack_tables_body,
        grid=(2 * nhalf,),
        in_specs=[
            pl.BlockSpec((2 * MRB, DIM),
                         lambda i: (jnp.minimum(i, nhalf - 1), 0)),
            pl.BlockSpec((2 * MRB, DIM),
                         lambda i: (jnp.maximum(i - nhalf, 0), 0)),
        ],
        out_specs=pl.BlockSpec((MRB, 2 * DIM), lambda i: (i, 0)),
        out_shape=jax.ShapeDtypeStruct(
            ((ENTITIES + RELATIONS) // 2, 2 * DIM), jnp.float32),
    )(entity_embeddings.astype(jnp.float32),
      relation_embeddings.astype(jnp.float32))
    merged = m2.reshape(ENTITIES + RELATIONS, DIM)

    ids = object_ids.reshape(-1).astype(jnp.int32)
    types = object_types.reshape(-1).astype(jnp.int32)

    mesh = plsc.VectorSubcoreMesh(core_axis_name="c", subcore_axis_name="s")
    run_sc = pl.kernel(
        _sc_body,
        out_type=jax.ShapeDtypeStruct((FLAT, DIM), jnp.float32),
        mesh=mesh,
        scratch_types=[
            pltpu.VMEM((PER_W,), jnp.int32),
            pltpu.VMEM((PER_W,), jnp.int32),
            pltpu.VMEM((CHUNK, DIM), jnp.float32),
            pltpu.VMEM((CHUNK, DIM), jnp.float32),
            pltpu.SemaphoreType.DMA,
            pltpu.SemaphoreType.DMA,
            pltpu.SemaphoreType.DMA,
            pltpu.SemaphoreType.DMA,
        ],
        compiler_params=pltpu.CompilerParams(use_tc_tiling_on_sc=False),
    )
    gathered = run_sc(merged, ids, types)

    in2 = gathered.reshape(FLAT // 2, 128)
    out2 = pl.pallas_call(
        _tc_body,
        grid=(B // TBB,),
        in_specs=[
            pl.BlockSpec((TBB * L // 2, 128), lambda i: (i, 0)),
            pl.BlockSpec((ROWD, 1), lambda i: (0, 0)),
        ],
        out_specs=pl.BlockSpec((ROWD, TBB), lambda i: (0, i)),
        out_shape=jax.ShapeDtypeStruct((ROWD, B), jnp.float32),
    )(in2, pe_col)

    return out2.reshape(L, DIM, B).transpose(2, 0, 1)
